# baseline (speedup 1.0000x reference)
import functools

import jax
import jax.numpy as jnp
import numpy as np
from jax import lax
from jax.experimental import pallas as pl
from jax.experimental.pallas import tpu as pltpu


def _conv(x, w, b, stride, pad):
    y = lax.conv_general_dilated(x, w, (stride, stride), ((pad, pad), (pad, pad)),
                                 dimension_numbers=('NCHW', 'OIHW', 'NCHW'))
    return y + b[None, :, None, None]


def _resblock(x, w1, b1, w2, b2):
    h = jax.nn.relu(x)
    h = _conv(h, w1, b1, 1, 1)
    h = jax.nn.relu(h)
    h = _conv(h, w2, b2, 1, 0)
    return x + h


def _encoder(x, p):
    x = jax.nn.relu(_conv(x, p['e_c1_w'], p['e_c1_b'], 2, 1))
    x = jax.nn.relu(_conv(x, p['e_c2_w'], p['e_c2_b'], 2, 1))
    x = _conv(x, p['e_c3_w'], p['e_c3_b'], 1, 1)
    for i in range(2):
        x = _resblock(x, p['e_rb%d_w1' % i], p['e_rb%d_b1' % i],
                      p['e_rb%d_w2' % i], p['e_rb%d_b2' % i])
    x = jax.nn.relu(x)
    x = _conv(x, p['e_out_w'], p['e_out_b'], 1, 0)
    return x


_NUM_CODES = 512
_P56 = 58
_N56 = _P56 * _P56
_HW = 56 * 56

_OFF3 = tuple((dy - 1) * _P56 + (dx - 1) for dy in range(3) for dx in range(3))
_OFF3_T = tuple(iy * _P56 + ix for iy in (-1, 0, 1) for ix in (-1, 0, 1))


def _shift_rows(p, s):
    if s == 0:
        return p
    n = p.shape[0]
    z = jnp.zeros((abs(s), p.shape[1]), p.dtype)
    if s > 0:
        return jnp.concatenate([p[s:], z], axis=0)
    return jnp.concatenate([z, p[:n + s]], axis=0)


def _im2col(x, offsets):
    xb = x.astype(jnp.bfloat16)
    return jnp.concatenate([_shift_rows(xb, s) for s in offsets], axis=1)


def _mm(a, b):
    return jax.lax.dot_general(a, b, (((1,), (0,)), ((), ())),
                               preferred_element_type=jnp.float32)


def _fused_body(x_ref, embt_ref, emb3_ref,
                wc1_ref, bc1_ref,
                rw1a_ref, rb1a_ref, rw2a_ref, rb2a_ref,
                rw1b_ref, rb1b_ref, rw2b_ref, rb2b_ref,
                wt1_ref, bt1_ref, wt2_ref, bt2_ref, mask_ref,
                out_ref, pad_ref):
    mask = mask_ref[...]
    x = jnp.transpose(x_ref[0])

    embt = embt_ref[...]
    m = _mm(x.astype(jnp.bfloat16), embt.astype(jnp.bfloat16))
    sx = jnp.sum(x * x, axis=1, keepdims=True)
    se = jnp.sum(embt * embt, axis=0, keepdims=True)
    d2 = (sx + se) - 2.0 * m
    dis = jnp.sqrt(jnp.maximum(d2, 0.0))
    minv = jnp.min(dis, axis=1, keepdims=True)
    iota = jax.lax.broadcasted_iota(jnp.int32, dis.shape, 1)
    idx = jnp.min(jnp.where(dis == minv, iota, _NUM_CODES), axis=1)
    onehot = (iota == idx[:, None]).astype(jnp.bfloat16)
    g = _mm(onehot, emb3_ref[...])
    q = (g[:, 0:64] + g[:, 64:128]) + g[:, 128:192]
    qx = x + (q - x)

    pad_ref[...] = jnp.zeros((_N56, 64), jnp.float32)
    for y in range(56):
        pad_ref[pl.ds((y + 1) * _P56 + 1, 56), :] = qx[y * 56:(y + 1) * 56, :]
    x0 = pad_ref[...]

    x1 = (_mm(_im2col(x0, _OFF3), wc1_ref[...]) + bc1_ref[...]) * mask
    for w1_ref, b1_ref, w2_ref, b2_ref in (
            (rw1a_ref, rb1a_ref, rw2a_ref, rb2a_ref),
            (rw1b_ref, rb1b_ref, rw2b_ref, rb2b_ref)):
        h = jax.nn.relu(x1)
        h = (_mm(_im2col(h, _OFF3), w1_ref[...]) + b1_ref[...]) * mask
        h = jax.nn.relu(h)
        h = _mm(h.astype(jnp.bfloat16), w2_ref[...]) + b2_ref[...]
        x1 = x1 + h * mask
    x1 = jax.nn.relu(x1)

    y = _mm(_im2col(x1, _OFF3_T), wt1_ref[...]) + bt1_ref[...]
    y = jax.nn.relu(y) * mask

    out = _mm(_im2col(y, _OFF3_T), wt2_ref[...]) + bt2_ref[...]
    out_ref[0] = out * mask


@jax.jit
def _fused_pallas(xf, embt, emb3, wc1, bc1, rws, wt1, bt1, wt2, bt2, mask):
    (rw1a, rb1a, rw2a, rb2a), (rw1b, rb1b, rw2b, rb2b) = rws
    args = [xf, embt, emb3, wc1, bc1, rw1a, rb1a, rw2a, rb2a,
            rw1b, rb1b, rw2b, rb2b, wt1, bt1, wt2, bt2, mask]
    in_specs = [pl.BlockSpec((1, 64, _HW), lambda i: (i, 0, 0))]
    for a in args[1:]:
        nd = len(a.shape)
        in_specs.append(pl.BlockSpec(a.shape, lambda i, _nd=nd: (0,) * _nd))
    return pl.pallas_call(
        _fused_body,
        grid=(xf.shape[0],),
        in_specs=in_specs,
        out_specs=pl.BlockSpec((1, _N56, 48), lambda i: (i, 0, 0)),
        out_shape=jax.ShapeDtypeStruct((xf.shape[0], _N56, 48), jnp.float32),
        scratch_shapes=[pltpu.VMEM((_N56, 64), jnp.float32)],
    )(*args)


def _taps3x3(w):
    co, ci = w.shape[0], w.shape[1]
    return w.transpose(2, 3, 1, 0).reshape(9 * ci, co).astype(jnp.bfloat16)


_CT_TAPS = {0: ((-1, 3), (0, 1)), 1: ((0, 2), (1, 0))}


def _t1_maps():
    ymap = {(py, iy): ky for py in (0, 1) for iy, ky in _CT_TAPS[py]}
    ky = np.zeros((9, 4), np.int32)
    kx = np.zeros((9, 4), np.int32)
    valid = np.zeros((9, 4), np.float32)
    for t, (iy, ix) in enumerate((iy, ix) for iy in (-1, 0, 1) for ix in (-1, 0, 1)):
        for ph, (py, px) in enumerate((py, px) for py in (0, 1) for px in (0, 1)):
            a = ymap.get((py, iy))
            b = ymap.get((px, ix))
            if a is not None and b is not None:
                ky[t, ph], kx[t, ph], valid[t, ph] = a, b, 1.0
    return ky, kx, valid


_T1_KY, _T1_KX, _T1_VALID = _t1_maps()


def _t1_weights(w):
    ci, co = w.shape[0], w.shape[1]
    g = w[:, :, jnp.asarray(_T1_KY), jnp.asarray(_T1_KX)]
    g = g * jnp.asarray(_T1_VALID)
    return g.transpose(2, 0, 3, 1).reshape(9 * ci, 4 * co).astype(jnp.bfloat16)


def _t2_maps():
    ymap = {}
    for py in (0, 1):
        for qy in (0, 1):
            for j, ky in _CT_TAPS[qy]:
                src = py + j
                ymap.setdefault((src // 2, src % 2), {})[(py, qy)] = ky
    ky = np.zeros((36, 16), np.int32)
    kx = np.zeros((36, 16), np.int32)
    valid = np.zeros((36, 16), np.float32)
    blocks = [(sy, sx, psy, psx) for sy in (-1, 0, 1) for sx in (-1, 0, 1)
              for psy in (0, 1) for psx in (0, 1)]
    phases = [(py, qy, px, qx) for py in (0, 1) for qy in (0, 1)
              for px in (0, 1) for qx in (0, 1)]
    for bi, (sy, sx, psy, psx) in enumerate(blocks):
        my = ymap.get((sy, psy), {})
        mx = ymap.get((sx, psx), {})
        for pi, (py, qy, px, qx) in enumerate(phases):
            a = my.get((py, qy))
            b = mx.get((px, qx))
            if a is not None and b is not None:
                ky[bi, pi], kx[bi, pi], valid[bi, pi] = a, b, 1.0
    return ky, kx, valid


_T2_KY, _T2_KX, _T2_VALID = _t2_maps()


def _t2_weights(w):
    ci, co = w.shape[0], w.shape[1]
    g = w[:, :, jnp.asarray(_T2_KY), jnp.asarray(_T2_KX)]
    g = g * jnp.asarray(_T2_VALID)
    return g.transpose(2, 0, 3, 1).reshape(36 * ci, 16 * co).astype(jnp.bfloat16)


def _emb_split3(emb):
    hi = emb.astype(jnp.bfloat16)
    r1 = emb - hi.astype(jnp.float32)
    mid = r1.astype(jnp.bfloat16)
    lo = (r1 - mid.astype(jnp.float32)).astype(jnp.bfloat16)
    return jnp.concatenate([hi, mid, lo], axis=1)


def _mask_plane():
    m = np.zeros((_P56, _P56, 1), np.float32)
    m[1:57, 1:57] = 1.0
    return m.reshape(_N56, 1)


_MASK56 = _mask_plane()


def kernel(img, params):
    p = params
    x = _encoder(img, p)
    B, C, H, W = x.shape
    xf = x.reshape(B, C, H * W)

    emb = p['embedding']
    rws = tuple((_taps3x3(p['d_rb%d_w1' % i]), p['d_rb%d_b1' % i][None, :],
                 p['d_rb%d_w2' % i][:, :, 0, 0].T.astype(jnp.bfloat16),
                 p['d_rb%d_b2' % i][None, :]) for i in range(2))
    out = _fused_pallas(
        xf, emb.T, _emb_split3(emb),
        _taps3x3(p['d_c1_w']), p['d_c1_b'][None, :], rws,
        _t1_weights(p['d_t1_w']), jnp.tile(p['d_t1_b'], 4)[None, :],
        _t2_weights(p['d_t2_w']), jnp.tile(p['d_t2_b'], 16)[None, :],
        jnp.asarray(_MASK56))

    pred = out.reshape(B, _P56, _P56, 4, 4, 3)[:, 1:57, 1:57]
    pred = pred.transpose(0, 5, 1, 3, 2, 4).reshape(B, 3, 224, 224)
    return pred


# speedup vs baseline: 1.1915x; 1.1915x over previous
import functools

import jax
import jax.numpy as jnp
import numpy as np
from jax import lax
from jax.experimental import pallas as pl
from jax.experimental.pallas import tpu as pltpu


def _conv(x, w, b, stride, pad):
    y = lax.conv_general_dilated(x, w, (stride, stride), ((pad, pad), (pad, pad)),
                                 dimension_numbers=('NCHW', 'OIHW', 'NCHW'))
    return y + b[None, :, None, None]


def _resblock(x, w1, b1, w2, b2):
    h = jax.nn.relu(x)
    h = _conv(h, w1, b1, 1, 1)
    h = jax.nn.relu(h)
    h = _conv(h, w2, b2, 1, 0)
    return x + h


def _encoder(x, p):
    x = jax.nn.relu(_conv(x, p['e_c1_w'], p['e_c1_b'], 2, 1))
    x = jax.nn.relu(_conv(x, p['e_c2_w'], p['e_c2_b'], 2, 1))
    x = _conv(x, p['e_c3_w'], p['e_c3_b'], 1, 1)
    for i in range(2):
        x = _resblock(x, p['e_rb%d_w1' % i], p['e_rb%d_b1' % i],
                      p['e_rb%d_w2' % i], p['e_rb%d_b2' % i])
    x = jax.nn.relu(x)
    x = _conv(x, p['e_out_w'], p['e_out_b'], 1, 0)
    return x


_NUM_CODES = 512
_P56 = 58
_N56 = _P56 * _P56
_HW = 56 * 56

_OFF3 = tuple((dy - 1) * _P56 + (dx - 1) for dy in range(3) for dx in range(3))
_OFF3_T = tuple(iy * _P56 + ix for iy in (-1, 0, 1) for ix in (-1, 0, 1))


def _shift_rows(p, s):
    if s == 0:
        return p
    n = p.shape[0]
    z = jnp.zeros((abs(s), p.shape[1]), p.dtype)
    if s > 0:
        return jnp.concatenate([p[s:], z], axis=0)
    return jnp.concatenate([z, p[:n + s]], axis=0)


def _im2col(x, offsets):
    xb = x.astype(jnp.bfloat16)
    return jnp.concatenate([_shift_rows(xb, s) for s in offsets], axis=1)


def _mm(a, b):
    return jax.lax.dot_general(a, b, (((1,), (0,)), ((), ())),
                               preferred_element_type=jnp.float32)


def _store_interior(pad_ref, v):
    pad_ref[pl.ds(1, 56), pl.ds(1, 56), :] = v.reshape(56, 56, v.shape[1])


def _im2col3d(pad_ref, blocks):
    xb = pad_ref[...]
    xs = [xb[:, dx:dx + 56, :] for dx in range(3)]
    cols = [xs[dx][dy:dy + 56, :, lo:hi] for dy, dx, lo, hi in blocks]
    c9 = jnp.concatenate(cols, axis=2)
    return c9.reshape(_HW, c9.shape[2])


_B3X3 = [(dy, dx, 0, None) for dy in range(3) for dx in range(3)]

_YC = [(-1, 1), (0, 0), (0, 1), (1, 0)]
_B_T2 = [(sy + 1, sx + 1, (psy * 2 + psx) * 64, (psy * 2 + psx) * 64 + 64)
         for (sy, psy) in _YC for (sx, psx) in _YC]


def _fused_body(x_ref, embt_ref, emb3_ref,
                wc1_ref, bc1_ref,
                rw1a_ref, rb1a_ref, rw2a_ref, rb2a_ref,
                rw1b_ref, rb1b_ref, rw2b_ref, rb2b_ref,
                wt1_ref, bt1_ref, wt2_ref, bt2_ref,
                out_ref, pad64_ref, pad128_ref, pad256_ref):
    x = x_ref[0]

    embt = embt_ref[...]
    m = _mm(x.astype(jnp.bfloat16), embt.astype(jnp.bfloat16))
    sx = jnp.sum(x * x, axis=1, keepdims=True)
    se = jnp.sum(embt * embt, axis=0, keepdims=True)
    d2 = (sx + se) - 2.0 * m
    dis = jnp.sqrt(jnp.maximum(d2, 0.0))
    minv = jnp.min(dis, axis=1, keepdims=True)
    iota = jax.lax.broadcasted_iota(jnp.int32, dis.shape, 1)
    idx = jnp.min(jnp.where(dis == minv, iota, _NUM_CODES), axis=1)
    onehot = (iota == idx[:, None]).astype(jnp.bfloat16)
    g = _mm(onehot, emb3_ref[...])
    q = (g[:, 0:64] + g[:, 64:128]) + g[:, 128:192]
    qx = x + (q - x)

    pad64_ref[...] = jnp.zeros(pad64_ref.shape, pad64_ref.dtype)
    pad128_ref[...] = jnp.zeros(pad128_ref.shape, pad128_ref.dtype)
    pad256_ref[...] = jnp.zeros(pad256_ref.shape, pad256_ref.dtype)

    _store_interior(pad64_ref, qx.astype(jnp.bfloat16))
    x1 = _mm(_im2col3d(pad64_ref, _B3X3), wc1_ref[...]) + bc1_ref[...]
    for w1_ref, b1_ref, w2_ref, b2_ref in (
            (rw1a_ref, rb1a_ref, rw2a_ref, rb2a_ref),
            (rw1b_ref, rb1b_ref, rw2b_ref, rb2b_ref)):
        _store_interior(pad128_ref, jax.nn.relu(x1).astype(jnp.bfloat16))
        h = _mm(_im2col3d(pad128_ref, _B3X3), w1_ref[...]) + b1_ref[...]
        h = jax.nn.relu(h)
        h = _mm(h.astype(jnp.bfloat16), w2_ref[...]) + b2_ref[...]
        x1 = x1 + h

    _store_interior(pad128_ref, jax.nn.relu(x1).astype(jnp.bfloat16))
    y = _mm(_im2col3d(pad128_ref, _B3X3), wt1_ref[...]) + bt1_ref[...]
    y = jax.nn.relu(y)

    _store_interior(pad256_ref, y.astype(jnp.bfloat16))
    out = _mm(_im2col3d(pad256_ref, _B_T2), wt2_ref[...]) + bt2_ref[...]
    out_ref[0] = out


@jax.jit
def _fused_pallas(xf, embt, emb3, wc1, bc1, rws, wt1, bt1, wt2, bt2):
    (rw1a, rb1a, rw2a, rb2a), (rw1b, rb1b, rw2b, rb2b) = rws
    args = [xf, embt, emb3, wc1, bc1, rw1a, rb1a, rw2a, rb2a,
            rw1b, rb1b, rw2b, rb2b, wt1, bt1, wt2, bt2]
    in_specs = [pl.BlockSpec((1, _HW, 64), lambda i: (i, 0, 0))]
    for a in args[1:]:
        nd = len(a.shape)
        in_specs.append(pl.BlockSpec(a.shape, lambda i, _nd=nd: (0,) * _nd))
    return pl.pallas_call(
        _fused_body,
        grid=(xf.shape[0],),
        in_specs=in_specs,
        out_specs=pl.BlockSpec((1, _HW, 48), lambda i: (i, 0, 0)),
        out_shape=jax.ShapeDtypeStruct((xf.shape[0], _HW, 48), jnp.float32),
        scratch_shapes=[pltpu.VMEM((_P56, _P56, 64), jnp.bfloat16),
                        pltpu.VMEM((_P56, _P56, 128), jnp.bfloat16),
                        pltpu.VMEM((_P56, _P56, 256), jnp.bfloat16)],
    )(*args)


def _taps3x3(w):
    co, ci = w.shape[0], w.shape[1]
    return w.transpose(2, 3, 1, 0).reshape(9 * ci, co).astype(jnp.bfloat16)


_CT_TAPS = {0: ((-1, 3), (0, 1)), 1: ((0, 2), (1, 0))}


def _t1_maps():
    ymap = {(py, iy): ky for py in (0, 1) for iy, ky in _CT_TAPS[py]}
    ky = np.zeros((9, 4), np.int32)
    kx = np.zeros((9, 4), np.int32)
    valid = np.zeros((9, 4), np.float32)
    for t, (iy, ix) in enumerate((iy, ix) for iy in (-1, 0, 1) for ix in (-1, 0, 1)):
        for ph, (py, px) in enumerate((py, px) for py in (0, 1) for px in (0, 1)):
            a = ymap.get((py, iy))
            b = ymap.get((px, ix))
            if a is not None and b is not None:
                ky[t, ph], kx[t, ph], valid[t, ph] = a, b, 1.0
    return ky, kx, valid


_T1_KY, _T1_KX, _T1_VALID = _t1_maps()


def _t1_weights(w):
    ci, co = w.shape[0], w.shape[1]
    g = w[:, :, jnp.asarray(_T1_KY), jnp.asarray(_T1_KX)]
    g = g * jnp.asarray(_T1_VALID)
    return g.transpose(2, 0, 3, 1).reshape(9 * ci, 4 * co).astype(jnp.bfloat16)


def _t2_maps():
    ymap = {}
    for py in (0, 1):
        for qy in (0, 1):
            for j, ky in _CT_TAPS[qy]:
                src = py + j
                ymap.setdefault((src // 2, src % 2), {})[(py, qy)] = ky
    ky = np.zeros((16, 16), np.int32)
    kx = np.zeros((16, 16), np.int32)
    valid = np.zeros((16, 16), np.float32)
    blocks = [(sy, psy, sx, psx) for (sy, psy) in _YC for (sx, psx) in _YC]
    phases = [(py, qy, px, qx) for py in (0, 1) for qy in (0, 1)
              for px in (0, 1) for qx in (0, 1)]
    for bi, (sy, psy, sx, psx) in enumerate(blocks):
        my = ymap.get((sy, psy), {})
        mx = ymap.get((sx, psx), {})
        for pi, (py, qy, px, qx) in enumerate(phases):
            a = my.get((py, qy))
            b = mx.get((px, qx))
            if a is not None and b is not None:
                ky[bi, pi], kx[bi, pi], valid[bi, pi] = a, b, 1.0
    return ky, kx, valid


_T2_KY, _T2_KX, _T2_VALID = _t2_maps()


def _t2_weights(w):
    ci, co = w.shape[0], w.shape[1]
    g = w[:, :, jnp.asarray(_T2_KY), jnp.asarray(_T2_KX)]
    g = g * jnp.asarray(_T2_VALID)
    return g.transpose(2, 0, 3, 1).reshape(16 * ci, 16 * co).astype(jnp.bfloat16)


def _emb_split3(emb):
    hi = emb.astype(jnp.bfloat16)
    r1 = emb - hi.astype(jnp.float32)
    mid = r1.astype(jnp.bfloat16)
    lo = (r1 - mid.astype(jnp.float32)).astype(jnp.bfloat16)
    return jnp.concatenate([hi, mid, lo], axis=1)


def kernel(img, params):
    p = params
    x = _encoder(img, p)
    B, C, H, W = x.shape
    xf = x.transpose(0, 2, 3, 1).reshape(B, H * W, C)

    emb = p['embedding']
    rws = tuple((_taps3x3(p['d_rb%d_w1' % i]), p['d_rb%d_b1' % i][None, :],
                 p['d_rb%d_w2' % i][:, :, 0, 0].T.astype(jnp.bfloat16),
                 p['d_rb%d_b2' % i][None, :]) for i in range(2))
    out = _fused_pallas(
        xf, emb.T, _emb_split3(emb),
        _taps3x3(p['d_c1_w']), p['d_c1_b'][None, :], rws,
        _t1_weights(p['d_t1_w']), jnp.tile(p['d_t1_b'], 4)[None, :],
        _t2_weights(p['d_t2_w']), jnp.tile(p['d_t2_b'], 16)[None, :])

    pred = out.reshape(B, 56, 56, 4, 4, 3)
    pred = pred.transpose(0, 5, 1, 3, 2, 4).reshape(B, 3, 224, 224)
    return pred


# speedup vs baseline: 1.2043x; 1.0107x over previous
import functools

import jax
import jax.numpy as jnp
import numpy as np
from jax import lax
from jax.experimental import pallas as pl
from jax.experimental.pallas import tpu as pltpu


def _conv(x, w, b, stride, pad):
    y = lax.conv_general_dilated(x, w, (stride, stride), ((pad, pad), (pad, pad)),
                                 dimension_numbers=('NCHW', 'OIHW', 'NCHW'))
    return y + b[None, :, None, None]


def _resblock(x, w1, b1, w2, b2):
    h = jax.nn.relu(x)
    h = _conv(h, w1, b1, 1, 1)
    h = jax.nn.relu(h)
    h = _conv(h, w2, b2, 1, 0)
    return x + h


def _encoder(x, p):
    x = jax.nn.relu(_conv(x, p['e_c1_w'], p['e_c1_b'], 2, 1))
    x = jax.nn.relu(_conv(x, p['e_c2_w'], p['e_c2_b'], 2, 1))
    x = _conv(x, p['e_c3_w'], p['e_c3_b'], 1, 1)
    for i in range(2):
        x = _resblock(x, p['e_rb%d_w1' % i], p['e_rb%d_b1' % i],
                      p['e_rb%d_w2' % i], p['e_rb%d_b2' % i])
    x = jax.nn.relu(x)
    x = _conv(x, p['e_out_w'], p['e_out_b'], 1, 0)
    return x


_NUM_CODES = 512
_P56 = 58
_N56 = _P56 * _P56
_HW = 56 * 56

_OFF3 = tuple((dy - 1) * _P56 + (dx - 1) for dy in range(3) for dx in range(3))
_OFF3_T = tuple(iy * _P56 + ix for iy in (-1, 0, 1) for ix in (-1, 0, 1))


def _shift_rows(p, s):
    if s == 0:
        return p
    n = p.shape[0]
    z = jnp.zeros((abs(s), p.shape[1]), p.dtype)
    if s > 0:
        return jnp.concatenate([p[s:], z], axis=0)
    return jnp.concatenate([z, p[:n + s]], axis=0)


def _im2col(x, offsets):
    xb = x.astype(jnp.bfloat16)
    return jnp.concatenate([_shift_rows(xb, s) for s in offsets], axis=1)


def _mm(a, b):
    return jax.lax.dot_general(a, b, (((1,), (0,)), ((), ())),
                               preferred_element_type=jnp.float32)


def _store_interior(pad_ref, v):
    pad_ref[pl.ds(1, 56), pl.ds(1, 56), :] = v.reshape(56, 56, v.shape[1])


def _im2col3d(pad_ref, blocks):
    xb = pad_ref[...]
    xs = [xb[:, dx:dx + 56, :] for dx in range(3)]
    cols = [xs[dx][dy:dy + 56, :, lo:hi] for dy, dx, lo, hi in blocks]
    c9 = jnp.concatenate(cols, axis=2)
    return c9.reshape(_HW, c9.shape[2])


_B3X3 = [(dy, dx, 0, None) for dy in range(3) for dx in range(3)]

_YC = [(-1, 1), (0, 0), (0, 1), (1, 0)]
_B_T2 = [(sy + 1, sx + 1, (psy * 2 + psx) * 64, (psy * 2 + psx) * 64 + 64)
         for (sy, psy) in _YC for (sx, psx) in _YC]


def _fused_body(x_ref, embt_ref, emb3_ref,
                wc1_ref, bc1_ref,
                rw1a_ref, rb1a_ref, rw2a_ref, rb2a_ref,
                rw1b_ref, rb1b_ref, rw2b_ref, rb2b_ref,
                wt1_ref, bt1_ref, wt2_ref, bt2_ref,
                out_ref, pad64_ref, pad128_ref, pad256_ref):
    x = x_ref[0]

    embt = embt_ref[...]
    m = _mm(x.astype(jnp.bfloat16), embt.astype(jnp.bfloat16))
    sx = jnp.sum(x * x, axis=1, keepdims=True)
    se = jnp.sum(embt * embt, axis=0, keepdims=True)
    d2 = (sx + se) - 2.0 * m
    dis = jnp.sqrt(jnp.maximum(d2, 0.0))
    minv = jnp.min(dis, axis=1, keepdims=True)
    iota = jax.lax.broadcasted_iota(jnp.int32, dis.shape, 1)
    idx = jnp.min(jnp.where(dis == minv, iota, _NUM_CODES), axis=1)
    onehot = (iota == idx[:, None]).astype(jnp.bfloat16)
    g = _mm(onehot, emb3_ref[...])
    q = (g[:, 0:64] + g[:, 64:128]) + g[:, 128:192]
    qx = x + (q - x)

    pad64_ref[...] = jnp.zeros(pad64_ref.shape, pad64_ref.dtype)
    pad128_ref[...] = jnp.zeros(pad128_ref.shape, pad128_ref.dtype)
    pad256_ref[...] = jnp.zeros(pad256_ref.shape, pad256_ref.dtype)

    _store_interior(pad64_ref, qx.astype(jnp.bfloat16))
    x1 = _mm(_im2col3d(pad64_ref, _B3X3), wc1_ref[...]) + bc1_ref[...]
    for w1_ref, b1_ref, w2_ref, b2_ref in (
            (rw1a_ref, rb1a_ref, rw2a_ref, rb2a_ref),
            (rw1b_ref, rb1b_ref, rw2b_ref, rb2b_ref)):
        _store_interior(pad128_ref, jax.nn.relu(x1).astype(jnp.bfloat16))
        h = _mm(_im2col3d(pad128_ref, _B3X3), w1_ref[...]) + b1_ref[...]
        h = jax.nn.relu(h)
        h = _mm(h.astype(jnp.bfloat16), w2_ref[...]) + b2_ref[...]
        x1 = x1 + h

    _store_interior(pad128_ref, jax.nn.relu(x1).astype(jnp.bfloat16))
    y = _mm(_im2col3d(pad128_ref, _B3X3), wt1_ref[...]) + bt1_ref[...]
    y = jax.nn.relu(y)

    _store_interior(pad256_ref, y.astype(jnp.bfloat16))
    out = _mm(_im2col3d(pad256_ref, _B_T2), wt2_ref[...]) + bt2_ref[...]
    out_ref[0] = jnp.transpose(out)


@jax.jit
def _fused_pallas(xf, embt, emb3, wc1, bc1, rws, wt1, bt1, wt2, bt2):
    (rw1a, rb1a, rw2a, rb2a), (rw1b, rb1b, rw2b, rb2b) = rws
    args = [xf, embt, emb3, wc1, bc1, rw1a, rb1a, rw2a, rb2a,
            rw1b, rb1b, rw2b, rb2b, wt1, bt1, wt2, bt2]
    in_specs = [pl.BlockSpec((1, _HW, 64), lambda i: (i, 0, 0))]
    for a in args[1:]:
        nd = len(a.shape)
        in_specs.append(pl.BlockSpec(a.shape, lambda i, _nd=nd: (0,) * _nd))
    return pl.pallas_call(
        _fused_body,
        grid=(xf.shape[0],),
        in_specs=in_specs,
        out_specs=pl.BlockSpec((1, 48, _HW), lambda i: (i, 0, 0)),
        out_shape=jax.ShapeDtypeStruct((xf.shape[0], 48, _HW), jnp.float32),
        scratch_shapes=[pltpu.VMEM((_P56, _P56, 64), jnp.bfloat16),
                        pltpu.VMEM((_P56, _P56, 128), jnp.bfloat16),
                        pltpu.VMEM((_P56, _P56, 256), jnp.bfloat16)],
    )(*args)


def _taps3x3(w):
    co, ci = w.shape[0], w.shape[1]
    return w.transpose(2, 3, 1, 0).reshape(9 * ci, co).astype(jnp.bfloat16)


_CT_TAPS = {0: ((-1, 3), (0, 1)), 1: ((0, 2), (1, 0))}


def _t1_maps():
    ymap = {(py, iy): ky for py in (0, 1) for iy, ky in _CT_TAPS[py]}
    ky = np.zeros((9, 4), np.int32)
    kx = np.zeros((9, 4), np.int32)
    valid = np.zeros((9, 4), np.float32)
    for t, (iy, ix) in enumerate((iy, ix) for iy in (-1, 0, 1) for ix in (-1, 0, 1)):
        for ph, (py, px) in enumerate((py, px) for py in (0, 1) for px in (0, 1)):
            a = ymap.get((py, iy))
            b = ymap.get((px, ix))
            if a is not None and b is not None:
                ky[t, ph], kx[t, ph], valid[t, ph] = a, b, 1.0
    return ky, kx, valid


_T1_KY, _T1_KX, _T1_VALID = _t1_maps()


def _t1_weights(w):
    ci, co = w.shape[0], w.shape[1]
    g = w[:, :, jnp.asarray(_T1_KY), jnp.asarray(_T1_KX)]
    g = g * jnp.asarray(_T1_VALID)
    return g.transpose(2, 0, 3, 1).reshape(9 * ci, 4 * co).astype(jnp.bfloat16)


def _t2_maps():
    ymap = {}
    for py in (0, 1):
        for qy in (0, 1):
            for j, ky in _CT_TAPS[qy]:
                src = py + j
                ymap.setdefault((src // 2, src % 2), {})[(py, qy)] = ky
    ky = np.zeros((16, 16), np.int32)
    kx = np.zeros((16, 16), np.int32)
    valid = np.zeros((16, 16), np.float32)
    blocks = [(sy, psy, sx, psx) for (sy, psy) in _YC for (sx, psx) in _YC]
    phases = [(py, qy, px, qx) for py in (0, 1) for qy in (0, 1)
              for px in (0, 1) for qx in (0, 1)]
    for bi, (sy, psy, sx, psx) in enumerate(blocks):
        my = ymap.get((sy, psy), {})
        mx = ymap.get((sx, psx), {})
        for pi, (py, qy, px, qx) in enumerate(phases):
            a = my.get((py, qy))
            b = mx.get((px, qx))
            if a is not None and b is not None:
                ky[bi, pi], kx[bi, pi], valid[bi, pi] = a, b, 1.0
    return ky, kx, valid


_T2_KY, _T2_KX, _T2_VALID = _t2_maps()


def _t2_weights(w):
    ci, co = w.shape[0], w.shape[1]
    g = w[:, :, jnp.asarray(_T2_KY), jnp.asarray(_T2_KX)]
    g = g * jnp.asarray(_T2_VALID)
    return g.transpose(2, 0, 3, 1).reshape(16 * ci, 16 * co).astype(jnp.bfloat16)


def _emb_split3(emb):
    hi = emb.astype(jnp.bfloat16)
    r1 = emb - hi.astype(jnp.float32)
    mid = r1.astype(jnp.bfloat16)
    lo = (r1 - mid.astype(jnp.float32)).astype(jnp.bfloat16)
    return jnp.concatenate([hi, mid, lo], axis=1)


def kernel(img, params):
    p = params
    x = _encoder(img, p)
    B, C, H, W = x.shape
    xf = x.transpose(0, 2, 3, 1).reshape(B, H * W, C)

    emb = p['embedding']
    rws = tuple((_taps3x3(p['d_rb%d_w1' % i]), p['d_rb%d_b1' % i][None, :],
                 p['d_rb%d_w2' % i][:, :, 0, 0].T.astype(jnp.bfloat16),
                 p['d_rb%d_b2' % i][None, :]) for i in range(2))
    out = _fused_pallas(
        xf, emb.T, _emb_split3(emb),
        _taps3x3(p['d_c1_w']), p['d_c1_b'][None, :], rws,
        _t1_weights(p['d_t1_w']), jnp.tile(p['d_t1_b'], 4)[None, :],
        _t2_weights(p['d_t2_w']), jnp.tile(p['d_t2_b'], 16)[None, :])

    pred = out.reshape(B, 4, 4, 3, 56, 56)
    pred = pred.transpose(0, 3, 4, 1, 5, 2).reshape(B, 3, 224, 224)
    return pred
